# baseline (speedup 1.0000x reference)
import functools

import jax
import jax.numpy as jnp
from jax import lax
from jax.experimental import pallas as pl
from jax.experimental.pallas import tpu as pltpu
from jax.experimental.pallas import tpu_sc as plsc

B = 4096
NF = 26
V = 100000
D = 16
H = 64
EPS = 1e-5

NC = 2
NS = 16
NW = NC * NS
BPW = B // NW
CHUNK = BPW * NF
GSZ = 128
NCH = CHUNK // GSZ


def _sc_gather(xi_hbm, offs_hbm, emb2_hbm, emb1_hbm, out2_hbm, out1_hbm,
               idx_v, offs_v, e2_v, e1_v, sem):
    wid = lax.axis_index("s") * NC + lax.axis_index("c")
    base = wid * CHUNK
    pltpu.sync_copy(xi_hbm.at[pl.ds(base, CHUNK)], idx_v)
    pltpu.sync_copy(offs_hbm, offs_v)
    for j in range(CHUNK // 16):
        sl = pl.ds(j * 16, 16)
        idx_v[sl] = idx_v[sl] + offs_v[sl]
    copies = []
    for ch in range(NCH):
        isl = idx_v.at[pl.ds(ch * GSZ, GSZ)]
        c2 = pltpu.make_async_copy(emb2_hbm.at[isl],
                                   e2_v.at[pl.ds(ch * GSZ, GSZ)], sem)
        c2.start()
        copies.append(c2)
        c1 = pltpu.make_async_copy(emb1_hbm.at[isl],
                                   e1_v.at[pl.ds(ch * GSZ, GSZ)], sem)
        c1.start()
        copies.append(c1)
    for c in copies:
        c.wait()
    pltpu.sync_copy(e2_v, out2_hbm.at[pl.ds(base, CHUNK)])
    pltpu.sync_copy(e1_v, out1_hbm.at[pl.ds(base, CHUNK)])


TB = 256


def _tc_body(g_ref, e1_ref, xv_ref, w1_ref, w2_ref, w3_ref,
             b1_ref, g1_ref, be1_ref, b2_ref, g2_ref, be2_ref,
             b3_ref, g3_ref, be3_ref, bias_ref, out_ref):
    G = g_ref[...]
    xv = xv_ref[...]
    e1 = e1_ref[...]
    col_f = lax.broadcasted_iota(jnp.int32, (NF, NF * D), 1) // D
    row_f = lax.broadcasted_iota(jnp.int32, (NF, NF * D), 0)
    E = (col_f == row_f).astype(jnp.float32)
    xv_wide = jnp.dot(xv, E, preferred_element_type=jnp.float32)
    S = G * xv_wide
    col_d = lax.broadcasted_iota(jnp.int32, (NF * D, D), 0) % D
    d_idx = lax.broadcasted_iota(jnp.int32, (NF * D, D), 1)
    P = (col_d == d_idx).astype(jnp.float32)
    sum_emb = jnp.dot(S, P, preferred_element_type=jnp.float32)
    sq_sum = jnp.dot(S * S, P, preferred_element_type=jnp.float32)
    fm2 = 0.5 * jnp.sum(sum_emb * sum_emb - sq_sum, axis=1)
    fm1 = jnp.sum(e1 * xv, axis=1)
    inv_std = 1.0 / (1.0 + EPS) ** 0.5
    x = S
    for w_ref, b_ref, gg_ref, be_ref in ((w1_ref, b1_ref, g1_ref, be1_ref),
                                         (w2_ref, b2_ref, g2_ref, be2_ref),
                                         (w3_ref, b3_ref, g3_ref, be3_ref)):
        z = jnp.dot(x, w_ref[...], preferred_element_type=jnp.float32)
        z = (z + b_ref[...]) * (inv_std * gg_ref[...]) + be_ref[...]
        x = jnp.maximum(z, 0.0)
    out_ref[...] = fm1 + fm2 + jnp.sum(x, axis=1) + bias_ref[0, 0]


def kernel(Xi, Xv, emb1, emb2, W1, b1, W2, b2, W3, b3,
           g1, be1, g2, be2, g3, be3, bias):
    idx_flat = Xi.reshape(B * NF).astype(jnp.int32)
    offs = (jnp.arange(CHUNK, dtype=jnp.int32) % NF) * V
    emb2_flat = emb2.reshape(NF * V, D)
    emb1_flat = emb1.reshape(NF * V)

    mesh = plsc.VectorSubcoreMesh(core_axis_name="c", subcore_axis_name="s")
    sc = functools.partial(
        pl.kernel,
        mesh=mesh,
        compiler_params=pltpu.CompilerParams(use_tc_tiling_on_sc=False),
        out_type=(jax.ShapeDtypeStruct((B * NF, D), jnp.float32),
                  jax.ShapeDtypeStruct((B * NF,), jnp.float32)),
        scratch_types=[
            pltpu.VMEM((CHUNK,), jnp.int32),
            pltpu.VMEM((CHUNK,), jnp.int32),
            pltpu.VMEM((CHUNK, D), jnp.float32),
            pltpu.VMEM((CHUNK,), jnp.float32),
            pltpu.SemaphoreType.DMA,
        ],
    )(_sc_gather)
    rows2, rows1 = sc(idx_flat, offs, emb2_flat, emb1_flat)

    G = rows2.reshape(B, NF * D)
    e1g = rows1.reshape(B, NF)

    grid = B // TB
    full = lambda shp: pl.BlockSpec(shp, lambda i: (0, 0))
    out = pl.pallas_call(
        _tc_body,
        grid=(grid,),
        in_specs=[
            pl.BlockSpec((TB, NF * D), lambda i: (i, 0)),
            pl.BlockSpec((TB, NF), lambda i: (i, 0)),
            pl.BlockSpec((TB, NF), lambda i: (i, 0)),
            full((NF * D, H)), full((H, H)), full((H, H)),
            full((1, H)), full((1, H)), full((1, H)),
            full((1, H)), full((1, H)), full((1, H)),
            full((1, H)), full((1, H)), full((1, H)),
            full((1, 1)),
        ],
        out_specs=pl.BlockSpec((TB,), lambda i: (i,)),
        out_shape=jax.ShapeDtypeStruct((B,), jnp.float32),
    )(G, e1g, Xv,
      W1.T, W2.T, W3.T,
      b1.reshape(1, H), g1.reshape(1, H), be1.reshape(1, H),
      b2.reshape(1, H), g2.reshape(1, H), be2.reshape(1, H),
      b3.reshape(1, H), g3.reshape(1, H), be3.reshape(1, H),
      bias.reshape(1, 1))
    return out


# speedup vs baseline: 2.2409x; 2.2409x over previous
import functools

import jax
import jax.numpy as jnp
from jax import lax
from jax.experimental import pallas as pl
from jax.experimental.pallas import tpu as pltpu
from jax.experimental.pallas import tpu_sc as plsc

B = 4096
NF = 26
V = 100000
D = 16
H = 64
EPS = 1e-5

NC = 2
NS = 16
NW = NC * NS
ROWS = NF * D
RPW = ROWS // NW
GSZ = 128
NCH = B // GSZ


def _sc_gather(xi_hbm, tab2_hbm, tab1_hbm, out2_hbm, out1_hbm,
               idx_v, row_v, sem):
    wid = lax.axis_index("s") * NC + lax.axis_index("c")

    def do_row(r, tab_hbm, out_hbm, f):
        pltpu.sync_copy(xi_hbm.at[pl.ds(pl.multiple_of(f * B, 8), B)], idx_v)
        base = pl.multiple_of(r * V, 8)
        for j in range(B // 16):
            sl = pl.ds(j * 16, 16)
            idx_v[sl] = idx_v[sl] + base
        copies = []
        for c in range(NCH):
            cp = pltpu.make_async_copy(
                tab_hbm.at[idx_v.at[pl.ds(c * GSZ, GSZ)]],
                row_v.at[pl.ds(c * GSZ, GSZ)], sem)
            cp.start()
            copies.append(cp)
        for cp in copies:
            cp.wait()
        pltpu.sync_copy(row_v, out_hbm.at[pl.ds(pl.multiple_of(r * B, 8), B)])

    def loop_body(i, carry):
        r = wid * RPW + i
        do_row(r, tab2_hbm, out2_hbm, lax.shift_right_logical(r, 4))
        return carry

    lax.fori_loop(0, RPW, loop_body, 0)

    @pl.when(wid < NF)
    def _():
        do_row(wid, tab1_hbm, out1_hbm, wid)


TBC = 512


def _tc_body(g_ref, e1_ref, xv_ref, w1_ref, w2_ref, w3_ref,
             b1_ref, g1_ref, be1_ref, b2_ref, g2_ref, be2_ref,
             b3_ref, g3_ref, be3_ref, bias_ref, out_ref):
    Gt = g_ref[...]
    xvt = xv_ref[...]
    e1t = e1_ref[...]
    row_f = lax.broadcasted_iota(jnp.int32, (ROWS, NF), 0) // D
    f_idx = lax.broadcasted_iota(jnp.int32, (ROWS, NF), 1)
    E = (row_f == f_idx).astype(jnp.float32)
    xv_wide = jnp.dot(E, xvt, preferred_element_type=jnp.float32)
    St = Gt * xv_wide
    d_idx = lax.broadcasted_iota(jnp.int32, (D, ROWS), 0)
    row_d = lax.broadcasted_iota(jnp.int32, (D, ROWS), 1) % D
    P = (d_idx == row_d).astype(jnp.float32)
    sum_emb = jnp.dot(P, St, preferred_element_type=jnp.float32)
    sq_sum = jnp.dot(P, St * St, preferred_element_type=jnp.float32)
    fm2 = 0.5 * jnp.sum(sum_emb * sum_emb - sq_sum, axis=0)
    fm1 = jnp.sum(e1t * xvt, axis=0)
    inv_std = 1.0 / (1.0 + EPS) ** 0.5
    x = St
    for w_ref, b_ref, gg_ref, be_ref in ((w1_ref, b1_ref, g1_ref, be1_ref),
                                         (w2_ref, b2_ref, g2_ref, be2_ref),
                                         (w3_ref, b3_ref, g3_ref, be3_ref)):
        z = jnp.dot(w_ref[...], x, preferred_element_type=jnp.float32)
        z = (z + b_ref[...]) * (inv_std * gg_ref[...]) + be_ref[...]
        x = jnp.maximum(z, 0.0)
    out_ref[...] = fm1 + fm2 + jnp.sum(x, axis=0) + bias_ref[0, 0]


def kernel(Xi, Xv, emb1, emb2, W1, b1, W2, b2, W3, b3,
           g1, be1, g2, be2, g3, be3, bias):
    xi_t = jnp.transpose(Xi, (1, 2, 0)).reshape(NF * B).astype(jnp.int32)
    tab2 = jnp.transpose(emb2, (0, 2, 1)).reshape(ROWS * V)
    tab1 = jnp.transpose(emb1, (0, 2, 1)).reshape(NF * V)
    xv_t = jnp.transpose(Xv)

    mesh = plsc.VectorSubcoreMesh(core_axis_name="c", subcore_axis_name="s")
    sc = functools.partial(
        pl.kernel,
        mesh=mesh,
        compiler_params=pltpu.CompilerParams(use_tc_tiling_on_sc=False),
        out_type=(jax.ShapeDtypeStruct((ROWS * B,), jnp.float32),
                  jax.ShapeDtypeStruct((NF * B,), jnp.float32)),
        scratch_types=[
            pltpu.VMEM((B,), jnp.int32),
            pltpu.VMEM((B,), jnp.float32),
            pltpu.SemaphoreType.DMA,
        ],
    )(_sc_gather)
    rows2, rows1 = sc(xi_t, tab2, tab1)

    Gt = rows2.reshape(ROWS, B)
    e1t = rows1.reshape(NF, B)

    grid = B // TBC
    full = lambda shp: pl.BlockSpec(shp, lambda i: (0, 0))
    out = pl.pallas_call(
        _tc_body,
        grid=(grid,),
        in_specs=[
            pl.BlockSpec((ROWS, TBC), lambda i: (0, i)),
            pl.BlockSpec((NF, TBC), lambda i: (0, i)),
            pl.BlockSpec((NF, TBC), lambda i: (0, i)),
            full((H, ROWS)), full((H, H)), full((H, H)),
            full((H, 1)), full((H, 1)), full((H, 1)),
            full((H, 1)), full((H, 1)), full((H, 1)),
            full((H, 1)), full((H, 1)), full((H, 1)),
            full((1, 1)),
        ],
        out_specs=pl.BlockSpec((TBC,), lambda i: (i,)),
        out_shape=jax.ShapeDtypeStruct((B,), jnp.float32),
    )(Gt, e1t, xv_t,
      W1, W2, W3,
      b1.reshape(H, 1), g1.reshape(H, 1), be1.reshape(H, 1),
      b2.reshape(H, 1), g2.reshape(H, 1), be2.reshape(H, 1),
      b3.reshape(H, 1), g3.reshape(H, 1), be3.reshape(H, 1),
      bias.reshape(1, 1))
    return out


# speedup vs baseline: 2.4823x; 1.1077x over previous
import functools

import jax
import jax.numpy as jnp
from jax import lax
from jax.experimental import pallas as pl
from jax.experimental.pallas import tpu as pltpu
from jax.experimental.pallas import tpu_sc as plsc

B = 4096
NF = 26
V = 100000
D = 16
H = 64
EPS = 1e-5

NC = 2
NS = 16
ROWS = NF * D
NB2 = ROWS // 8
NB2C = NB2 // NC
NF_PAD = 32
NB1C = NF_PAD // 8 // NC
HB = B // 2
GSZ = 128
CHUNK_COLS = 6272
COL_OFF = [t * CHUNK_COLS for t in range(NS)]
COL_SZ = [CHUNK_COLS] * (NS - 1) + [V - (NS - 1) * CHUNK_COLS]


def _sc_gather(xi_hbm, tab2_hbm, tab1_hbm, out2_hbm, out1_hbm,
               buf0, buf1, idx_v, val_v, fsem0, fsem1, gsem):
    cid = lax.axis_index("c")
    sid = lax.axis_index("s")
    r = lax.div(sid, 2)
    h = lax.rem(sid, 2)
    bufs = (buf0, buf1)
    fsems = (fsem0, fsem1)
    co, cs = COL_OFF[0], COL_SZ[0]

    def fill(tab, q, buf, fsem):
        for t in range(NS):
            @pl.when(sid == t)
            def _():
                cp = pltpu.make_async_copy(
                    tab.at[q, :, pl.ds(COL_OFF[t], COL_SZ[t])],
                    buf.at[:, pl.ds(COL_OFF[t], COL_SZ[t])], fsem)
                cp.start()

    def wait_fill(buf, fsem):
        for t in range(NS):
            @pl.when(sid == t)
            def _():
                pltpu.make_async_copy(
                    buf.at[:, pl.ds(COL_OFF[t], COL_SZ[t])],
                    buf.at[:, pl.ds(COL_OFF[t], COL_SZ[t])], fsem).wait()

    def gather(buf, f, grow):
        off = pl.multiple_of(f * B + h * HB, 8)
        pltpu.sync_copy(xi_hbm.at[pl.ds(off, HB)], idx_v)
        copies = []
        for c in range(HB // GSZ):
            cp = pltpu.make_async_copy(
                buf.at[r].at[idx_v.at[pl.ds(c * GSZ, GSZ)]],
                val_v.at[pl.ds(c * GSZ, GSZ)], gsem)
            cp.start()
            copies.append(cp)
        for cp in copies:
            cp.wait()

    def write(out_hbm, grow):
        woff = pl.multiple_of(grow * B + h * HB, 8)
        pltpu.sync_copy(val_v, out_hbm.at[pl.ds(woff, HB)])

    base2 = cid * NB2C
    fill(tab2_hbm, base2, buf0, fsem0)

    def outer(i, carry):
        for b in range(2):
            k = 2 * i + b
            q = base2 + k
            wait_fill(bufs[b], fsems[b])
            plsc.subcore_barrier()

            @pl.when(k + 1 < NB2C)
            def _():
                fill(tab2_hbm, q + 1, bufs[1 - b], fsems[1 - b])

            grow = q * 8 + r
            gather(bufs[b], lax.div(grow, D), grow)
            write(out2_hbm, grow)
        return carry

    lax.fori_loop(0, NB2C // 2, outer, 0)

    plsc.subcore_barrier()
    base1 = cid * NB1C
    fill(tab1_hbm, base1, buf0, fsem0)
    for j in range(NB1C):
        qq = base1 + j
        wait_fill(bufs[j % 2], fsems[j % 2])
        plsc.subcore_barrier()
        if j + 1 < NB1C:
            fill(tab1_hbm, qq + 1, bufs[(j + 1) % 2], fsems[(j + 1) % 2])
        f1 = qq * 8 + r

        @pl.when(f1 < NF)
        def _():
            gather(bufs[j % 2], f1, f1)
            write(out1_hbm, f1)


TBC = 512


def _tc_body(g_ref, e1_ref, xv_ref, w1_ref, w2_ref, w3_ref,
             b1_ref, g1_ref, be1_ref, b2_ref, g2_ref, be2_ref,
             b3_ref, g3_ref, be3_ref, bias_ref, out_ref):
    Gt = g_ref[...]
    xvt = xv_ref[...]
    e1t = e1_ref[...]
    row_f = lax.broadcasted_iota(jnp.int32, (ROWS, NF), 0) // D
    f_idx = lax.broadcasted_iota(jnp.int32, (ROWS, NF), 1)
    E = (row_f == f_idx).astype(jnp.float32)
    xv_wide = jnp.dot(E, xvt, preferred_element_type=jnp.float32)
    St = Gt * xv_wide
    d_idx = lax.broadcasted_iota(jnp.int32, (D, ROWS), 0)
    row_d = lax.broadcasted_iota(jnp.int32, (D, ROWS), 1) % D
    P = (d_idx == row_d).astype(jnp.float32)
    sum_emb = jnp.dot(P, St, preferred_element_type=jnp.float32)
    sq_sum = jnp.dot(P, St * St, preferred_element_type=jnp.float32)
    fm2 = 0.5 * jnp.sum(sum_emb * sum_emb - sq_sum, axis=0)
    fm1 = jnp.sum(e1t * xvt, axis=0)
    inv_std = 1.0 / (1.0 + EPS) ** 0.5
    x = St
    for w_ref, b_ref, gg_ref, be_ref in ((w1_ref, b1_ref, g1_ref, be1_ref),
                                         (w2_ref, b2_ref, g2_ref, be2_ref),
                                         (w3_ref, b3_ref, g3_ref, be3_ref)):
        z = jnp.dot(w_ref[...], x, preferred_element_type=jnp.float32)
        z = (z + b_ref[...]) * (inv_std * gg_ref[...]) + be_ref[...]
        x = jnp.maximum(z, 0.0)
    out_ref[...] = fm1 + fm2 + jnp.sum(x, axis=0) + bias_ref[0, 0]


def kernel(Xi, Xv, emb1, emb2, W1, b1, W2, b2, W3, b3,
           g1, be1, g2, be2, g3, be3, bias):
    xi_t = jnp.transpose(Xi, (1, 2, 0)).reshape(NF * B).astype(jnp.int32)
    tab2 = jnp.transpose(emb2, (0, 2, 1)).reshape(NB2, 8, V)
    tab1 = jnp.pad(emb1[:, :, 0], ((0, NF_PAD - NF), (0, 0))).reshape(
        NF_PAD // 8, 8, V)
    xv_t = jnp.transpose(Xv)

    mesh = plsc.VectorSubcoreMesh(core_axis_name="c", subcore_axis_name="s")
    sc = functools.partial(
        pl.kernel,
        mesh=mesh,
        compiler_params=pltpu.CompilerParams(use_tc_tiling_on_sc=False),
        out_type=(jax.ShapeDtypeStruct((ROWS * B,), jnp.float32),
                  jax.ShapeDtypeStruct((NF * B,), jnp.float32)),
        scratch_types=[
            pltpu.VMEM_SHARED((8, V), jnp.float32),
            pltpu.VMEM_SHARED((8, V), jnp.float32),
            pltpu.VMEM((HB,), jnp.int32),
            pltpu.VMEM((HB,), jnp.float32),
            pltpu.SemaphoreType.DMA,
            pltpu.SemaphoreType.DMA,
            pltpu.SemaphoreType.DMA,
        ],
    )(_sc_gather)
    rows2, rows1 = sc(xi_t, tab2, tab1)

    Gt = rows2.reshape(ROWS, B)
    e1t = rows1.reshape(NF, B)

    grid = B // TBC
    full = lambda shp: pl.BlockSpec(shp, lambda i: (0, 0))
    out = pl.pallas_call(
        _tc_body,
        grid=(grid,),
        in_specs=[
            pl.BlockSpec((ROWS, TBC), lambda i: (0, i)),
            pl.BlockSpec((NF, TBC), lambda i: (0, i)),
            pl.BlockSpec((NF, TBC), lambda i: (0, i)),
            full((H, ROWS)), full((H, H)), full((H, H)),
            full((H, 1)), full((H, 1)), full((H, 1)),
            full((H, 1)), full((H, 1)), full((H, 1)),
            full((H, 1)), full((H, 1)), full((H, 1)),
            full((1, 1)),
        ],
        out_specs=pl.BlockSpec((TBC,), lambda i: (i,)),
        out_shape=jax.ShapeDtypeStruct((B,), jnp.float32),
    )(Gt, e1t, xv_t,
      W1, W2, W3,
      b1.reshape(H, 1), g1.reshape(H, 1), be1.reshape(H, 1),
      b2.reshape(H, 1), g2.reshape(H, 1), be2.reshape(H, 1),
      b3.reshape(H, 1), g3.reshape(H, 1), be3.reshape(H, 1),
      bias.reshape(1, 1))
    return out
